# baseline (speedup 1.0000x reference)
import functools

import jax
import jax.numpy as jnp
from jax import lax
from jax.experimental import pallas as pl
from jax.experimental.pallas import tpu as pltpu
from jax.experimental.pallas import tpu_sc as plsc

NC = 2
NS = 16


NBUF = 3


def _make_sc_scatter(E, N, D, chunk):
    assert E % NS == 0
    e_per_tile = E // NS
    assert chunk % 8 == 0 and chunk <= 128
    nchunks = e_per_tile // chunk
    tail = e_per_tile - nchunks * chunk
    assert tail % 8 == 0 and tail < chunk
    ngroups = nchunks // NBUF
    nrem = nchunks - ngroups * NBUF
    assert nchunks >= 2 * NBUF
    n_main = (N // (NS * 8)) * 8
    n_rem = N - NS * n_main
    assert n_rem % 8 == 0

    mesh = plsc.VectorSubcoreMesh(core_axis_name="c", subcore_axis_name="s")

    @functools.partial(
        pl.kernel,
        out_type=(
            jax.ShapeDtypeStruct((N, D), jnp.float32),
            jax.ShapeDtypeStruct((N, D), jnp.float32),
        ),
        mesh=mesh,
        scratch_types=[
            pltpu.VMEM_SHARED((N, D), jnp.float32),
            tuple(pltpu.VMEM((chunk,), jnp.int32) for _ in range(NBUF)),
            tuple(pltpu.VMEM((chunk, D), jnp.float32) for _ in range(NBUF)),
            tuple(pltpu.SemaphoreType.DMA for _ in range(NBUF)),
            tuple(pltpu.SemaphoreType.DMA for _ in range(NBUF)),
            pltpu.VMEM((tail if tail else 8,), jnp.int32),
        ],
    )
    def sc_scatter(idx_m_hbm, rows_m_hbm, idx_w_hbm, rows_w_hbm, zeros_hbm,
                   aggm_hbm, aggw_hbm, acc, idx_v, rows_v, lsems, ssems,
                   idx_tail):
        c = lax.axis_index("c")
        s = lax.axis_index("s")
        nbase = s * n_main
        ebase = s * e_per_tile

        def zero_init():
            pltpu.sync_copy(zeros_hbm.at[pl.ds(0, n_main)],
                            acc.at[pl.ds(nbase, n_main)])
            if n_rem:
                @pl.when(s == 0)
                def _():
                    pltpu.sync_copy(zeros_hbm.at[pl.ds(n_main, n_rem)],
                                    acc.at[pl.ds(NS * n_main, n_rem)])

        def write_back(out_hbm):
            pltpu.sync_copy(acc.at[pl.ds(nbase, n_main)],
                            out_hbm.at[pl.ds(nbase, n_main)])
            if n_rem:
                @pl.when(s == 0)
                def _():
                    pltpu.sync_copy(acc.at[pl.ds(NS * n_main, n_rem)],
                                    out_hbm.at[pl.ds(NS * n_main, n_rem)])

        def run(idx_hbm, rows_hbm, out_hbm):
            def start_load(k, b):
                off = E + ebase + k * chunk
                pltpu.async_copy(idx_hbm.at[pl.ds(off, chunk)], idx_v[b],
                                 lsems[b])
                pltpu.async_copy(rows_hbm.at[pl.ds(off, chunk)], rows_v[b],
                                 lsems[b])

            def wait_load(b):
                pltpu.make_async_copy(idx_hbm.at[pl.ds(0, chunk)], idx_v[b],
                                      lsems[b]).wait()
                pltpu.make_async_copy(rows_hbm.at[pl.ds(0, chunk)], rows_v[b],
                                      lsems[b]).wait()

            def start_scatter(b):
                pltpu.async_copy(rows_v[b], acc.at[idx_v[b]], ssems[b],
                                 add=True)

            def wait_scatter(b):
                pltpu.make_async_copy(rows_v[b], acc.at[idx_v[b]],
                                      ssems[b]).wait()

            start_load(0, 0)
            start_load(1, 1)
            tb = NBUF - 1
            if tail:
                toff = ebase + nchunks * chunk
                pltpu.async_copy(idx_hbm.at[pl.ds(E + toff, tail)], idx_tail,
                                 lsems[tb])
                pltpu.async_copy(rows_hbm.at[pl.ds(toff, tail)],
                                 rows_v[tb].at[pl.ds(0, tail)], lsems[tb])
            zero_init()
            plsc.subcore_barrier()
            if tail:
                pltpu.make_async_copy(idx_hbm.at[pl.ds(0, tail)], idx_tail,
                                      lsems[tb]).wait()
                pltpu.make_async_copy(rows_hbm.at[pl.ds(0, tail)],
                                      rows_v[tb].at[pl.ds(0, tail)],
                                      lsems[tb]).wait()
                pltpu.sync_copy(rows_v[tb].at[pl.ds(0, tail)],
                                acc.at[idx_tail], add=True)

            def group(g, carry):
                for b in range(NBUF):
                    k = g * NBUF + b
                    wait_load(b)
                    start_scatter(b)
                    nb = (b + 2) % NBUF

                    @pl.when(k + 2 < nchunks)
                    def _():
                        @pl.when(k >= NBUF - 2)
                        def _():
                            wait_scatter(nb)

                        start_load(k + 2, nb)
                return carry

            lax.fori_loop(0, ngroups, group, 0)
            for j in range(nrem):
                k = ngroups * NBUF + j
                b = k % NBUF
                wait_load(b)
                start_scatter(b)
            for b in range(NBUF):
                wait_scatter(b)
            plsc.subcore_barrier()
            write_back(out_hbm)

        @pl.when(c == 0)
        def _():
            run(idx_m_hbm, rows_m_hbm, aggm_hbm)

        @pl.when(c == 1)
        def _():
            run(idx_w_hbm, rows_w_hbm, aggw_hbm)

    return sc_scatter


def _mm1_body(n_ref, w0_ref, b_ref, o_ref):
    o_ref[...] = jnp.dot(n_ref[...], w0_ref[...],
                         preferred_element_type=jnp.float32) + b_ref[...]


def _mm2_body(y_ref, m_ref, w_ref, w1_ref, w2_ref, o_ref):
    acc = y_ref[...]
    acc += jnp.dot(m_ref[...], w1_ref[...], preferred_element_type=jnp.float32)
    acc += jnp.dot(w_ref[...], w2_ref[...], preferred_element_type=jnp.float32)
    o_ref[...] = acc


def _tc_matmul1(node, w0, b2d, bm):
    N, D = node.shape
    row_spec = pl.BlockSpec((bm, D), lambda i: (i, 0))
    full = pl.BlockSpec((D, D), lambda i: (0, 0))
    return pl.pallas_call(
        _mm1_body,
        grid=(N // bm,),
        in_specs=[row_spec, full, pl.BlockSpec((1, D), lambda i: (0, 0))],
        out_specs=row_spec,
        out_shape=jax.ShapeDtypeStruct((N, D), jnp.float32),
    )(node, w0, b2d)


def _tc_matmul2(y0, aggm, aggw, w1, w2, bm):
    N, D = y0.shape
    row_spec = pl.BlockSpec((bm, D), lambda i: (i, 0))
    full = pl.BlockSpec((D, D), lambda i: (0, 0))
    return pl.pallas_call(
        _mm2_body,
        grid=(N // bm,),
        in_specs=[row_spec, row_spec, row_spec, full, full],
        out_specs=row_spec,
        out_shape=jax.ShapeDtypeStruct((N, D), jnp.float32),
    )(y0, aggm, aggw, w1, w2)


def kernel(node_attr, edge_index, edge_attr, edge_world_index, edge_world_attr, W, b):
    N, D = node_attr.shape
    E = edge_attr.shape[0]

    receivers_m = edge_index.astype(jnp.int32).reshape(2 * E)
    receivers_w = edge_world_index.astype(jnp.int32).reshape(2 * E)

    n_main = (N // (NS * 8)) * 8
    zeros = jnp.zeros((n_main + (N - NS * n_main), D), jnp.float32)

    sc_scatter = _make_sc_scatter(E, N, D, chunk=128)
    aggm, aggw = sc_scatter(receivers_m, edge_attr, receivers_w,
                            edge_world_attr, zeros)

    w0 = W[0:D]
    w1 = W[D:2 * D]
    w2 = W[2 * D:3 * D]
    b2d = b.reshape(1, D)
    y0 = _tc_matmul1(node_attr, w0, b2d, bm=2000)
    return _tc_matmul2(y0, aggm, aggw, w1, w2, bm=2000)


# speedup vs baseline: 1.1197x; 1.1197x over previous
import functools

import jax
import jax.numpy as jnp
from jax import lax
from jax.experimental import pallas as pl
from jax.experimental.pallas import tpu as pltpu
from jax.experimental.pallas import tpu_sc as plsc

NC = 2
NS = 16


NBUF = 3


def _make_sc_scatter(E, N, D, chunk):
    assert E % NS == 0
    e_per_tile = E // NS
    assert chunk % 8 == 0 and chunk <= 128
    nchunks = e_per_tile // chunk
    tail = e_per_tile - nchunks * chunk
    assert tail % 8 == 0 and tail < chunk
    ngroups = nchunks // NBUF
    nrem = nchunks - ngroups * NBUF
    assert nchunks >= 2 * NBUF
    n_main = (N // (NS * 8)) * 8
    n_rem = N - NS * n_main
    assert n_rem % 8 == 0

    mesh = plsc.VectorSubcoreMesh(core_axis_name="c", subcore_axis_name="s")

    @functools.partial(
        pl.kernel,
        out_type=(
            jax.ShapeDtypeStruct((N, D), jnp.float32),
            jax.ShapeDtypeStruct((N, D), jnp.float32),
        ),
        mesh=mesh,
        scratch_types=[
            pltpu.VMEM_SHARED((N, D), jnp.float32),
            tuple(pltpu.VMEM((chunk,), jnp.int32) for _ in range(NBUF)),
            tuple(pltpu.VMEM((chunk, D), jnp.float32) for _ in range(NBUF)),
            tuple(pltpu.SemaphoreType.DMA for _ in range(NBUF)),
            tuple(pltpu.SemaphoreType.DMA for _ in range(NBUF)),
            pltpu.VMEM((tail if tail else 8,), jnp.int32),
        ],
    )
    def sc_scatter(idx_m_hbm, rows_m_hbm, idx_w_hbm, rows_w_hbm, zeros_hbm,
                   aggm_hbm, aggw_hbm, acc, idx_v, rows_v, lsems, ssems,
                   idx_tail):
        c = lax.axis_index("c")
        s = lax.axis_index("s")
        nbase = s * n_main
        ebase = s * e_per_tile

        def zero_init():
            pltpu.sync_copy(zeros_hbm.at[pl.ds(0, n_main)],
                            acc.at[pl.ds(nbase, n_main)])
            if n_rem:
                @pl.when(s == 0)
                def _():
                    pltpu.sync_copy(zeros_hbm.at[pl.ds(n_main, n_rem)],
                                    acc.at[pl.ds(NS * n_main, n_rem)])

        def write_back(out_hbm):
            pltpu.sync_copy(acc.at[pl.ds(nbase, n_main)],
                            out_hbm.at[pl.ds(nbase, n_main)])
            if n_rem:
                @pl.when(s == 0)
                def _():
                    pltpu.sync_copy(acc.at[pl.ds(NS * n_main, n_rem)],
                                    out_hbm.at[pl.ds(NS * n_main, n_rem)])

        def run(idx_hbm, rows_hbm, out_hbm):
            def start_load(k, b):
                off = E + ebase + k * chunk
                pltpu.async_copy(idx_hbm.at[pl.ds(off, chunk)], idx_v[b],
                                 lsems[b])
                pltpu.async_copy(rows_hbm.at[pl.ds(off, chunk)], rows_v[b],
                                 lsems[b])

            def wait_load(b):
                pltpu.make_async_copy(idx_hbm.at[pl.ds(0, chunk)], idx_v[b],
                                      lsems[b]).wait()
                pltpu.make_async_copy(rows_hbm.at[pl.ds(0, chunk)], rows_v[b],
                                      lsems[b]).wait()

            def start_scatter(b):
                pltpu.async_copy(rows_v[b], acc.at[idx_v[b]], ssems[b],
                                 add=True)

            def wait_scatter(b):
                pltpu.make_async_copy(rows_v[b], acc.at[idx_v[b]],
                                      ssems[b]).wait()

            start_load(0, 0)
            start_load(1, 1)
            tb = NBUF - 1
            if tail:
                toff = ebase + nchunks * chunk
                pltpu.async_copy(idx_hbm.at[pl.ds(E + toff, tail)], idx_tail,
                                 lsems[tb])
                pltpu.async_copy(rows_hbm.at[pl.ds(toff, tail)],
                                 rows_v[tb].at[pl.ds(0, tail)], lsems[tb])
            zero_init()
            plsc.subcore_barrier()
            if tail:
                pltpu.make_async_copy(idx_hbm.at[pl.ds(0, tail)], idx_tail,
                                      lsems[tb]).wait()
                pltpu.make_async_copy(rows_hbm.at[pl.ds(0, tail)],
                                      rows_v[tb].at[pl.ds(0, tail)],
                                      lsems[tb]).wait()
                pltpu.sync_copy(rows_v[tb].at[pl.ds(0, tail)],
                                acc.at[idx_tail], add=True)

            def group(g, carry):
                for b in range(NBUF):
                    k = g * NBUF + b
                    nb = (b + 2) % NBUF

                    @pl.when(k + 2 < nchunks)
                    def _():
                        @pl.when(k >= NBUF - 2)
                        def _():
                            wait_scatter(nb)

                        start_load(k + 2, nb)

                    wait_load(b)
                    start_scatter(b)
                return carry

            lax.fori_loop(0, ngroups, group, 0)
            for j in range(nrem):
                k = ngroups * NBUF + j
                b = k % NBUF
                wait_load(b)
                start_scatter(b)
            for b in range(NBUF):
                wait_scatter(b)
            plsc.subcore_barrier()
            write_back(out_hbm)

        @pl.when(c == 0)
        def _():
            run(idx_m_hbm, rows_m_hbm, aggm_hbm)

        @pl.when(c == 1)
        def _():
            run(idx_w_hbm, rows_w_hbm, aggw_hbm)

    return sc_scatter


def _mm_body(n_ref, m_ref, w_ref, w0_ref, w1_ref, w2_ref, b_ref, o_ref):
    acc = jnp.dot(n_ref[...], w0_ref[...], preferred_element_type=jnp.float32)
    acc += jnp.dot(m_ref[...], w1_ref[...], preferred_element_type=jnp.float32)
    acc += jnp.dot(w_ref[...], w2_ref[...], preferred_element_type=jnp.float32)
    o_ref[...] = acc + b_ref[...]


def _tc_matmul(node, aggm, aggw, w0, w1, w2, b2d, bm):
    N, D = node.shape
    row_spec = pl.BlockSpec((bm, D), lambda i: (i, 0))
    full = pl.BlockSpec((D, D), lambda i: (0, 0))
    return pl.pallas_call(
        _mm_body,
        grid=(N // bm,),
        in_specs=[row_spec, row_spec, row_spec, full, full, full,
                  pl.BlockSpec((1, D), lambda i: (0, 0))],
        out_specs=row_spec,
        out_shape=jax.ShapeDtypeStruct((N, D), jnp.float32),
    )(node, aggm, aggw, w0, w1, w2, b2d)


def kernel(node_attr, edge_index, edge_attr, edge_world_index, edge_world_attr, W, b):
    N, D = node_attr.shape
    E = edge_attr.shape[0]

    receivers_m = edge_index.astype(jnp.int32).reshape(2 * E)
    receivers_w = edge_world_index.astype(jnp.int32).reshape(2 * E)

    n_main = (N // (NS * 8)) * 8
    zeros = jnp.zeros((n_main + (N - NS * n_main), D), jnp.float32)

    sc_scatter = _make_sc_scatter(E, N, D, chunk=128)
    aggm, aggw = sc_scatter(receivers_m, edge_attr, receivers_w,
                            edge_world_attr, zeros)

    w0 = W[0:D]
    w1 = W[D:2 * D]
    w2 = W[2 * D:3 * D]
    b2d = b.reshape(1, D)
    return _tc_matmul(node_attr, aggm, aggw, w0, w1, w2, b2d, bm=2000)
